# baseline (speedup 1.0000x reference)
import functools

import jax
import jax.numpy as jnp
from jax import lax
from jax.experimental import pallas as pl
from jax.experimental.pallas import tpu as pltpu
from jax.experimental.pallas import tpu_sc as plsc

N = 10000
E = 320000
D = 128
DE = 16
INV_MAX_NEIGHBOR = 1.0 / 32.0
EPS = 1e-5

NC = 2
NS = 16
NW = NC * NS
NPAD = 10240
ROWS_PER_SUB = NPAD // NS

NHALF = 2
EH = E // NHALF
EPW = EH // NW
CHUNK = 40
NG = EPW // CHUNK
NB = 8
LAG = 4

_sc_mesh = plsc.VectorSubcoreMesh(core_axis_name="c", subcore_axis_name="s")


def _run_gather_phase(table_hbm, dst_hbm, idx_v, bufs, gsems, ssems, base):
    def idx_at(t):
        return idx_v.at[pl.ds(pl.multiple_of(t * CHUNK, 8), CHUNK)]

    def g_desc(t, b):
        return pltpu.make_async_copy(table_hbm.at[idx_at(t)], bufs.at[b],
                                     gsems.at[b])

    def s_desc(t, b):
        off = pl.multiple_of(base + t * CHUNK, 8)
        return pltpu.make_async_copy(bufs.at[b], dst_hbm.at[pl.ds(off, CHUNK)],
                                     ssems.at[b])

    def slot(t, b, first_round):
        if not first_round:
            s_desc(t - NB, b).wait()
        g_desc(t, b).start()
        if not (first_round and b < LAG):
            tl, bl = t - LAG, (b - LAG) % NB
            g_desc(tl, bl).wait()
            s_desc(tl, bl).start()

    for t in range(NB):
        slot(t, t, True)

    def outer(o, carry):
        for b in range(NB):
            slot(o * NB + b, b, False)
        return carry

    lax.fori_loop(1, NG // NB, outer, 0)

    for t in range((NG // NB) * NB, NG):
        slot(t, t % NB, False)
    for t in range(NG - LAG, NG):
        g_desc(t, t % NB).wait()
        s_desc(t, t % NB).start()
    for t in range(NG - NB, NG):
        s_desc(t, t % NB).wait()


def _make_gather(e0):
    @functools.partial(
        pl.kernel,
        out_type=(
            jax.ShapeDtypeStruct((EH, D), jnp.float32),
            jax.ShapeDtypeStruct((EH, D), jnp.float32),
        ),
        mesh=_sc_mesh,
        scratch_types=[
            pltpu.VMEM((EPW,), jnp.int32),
            pltpu.VMEM((NB, CHUNK, D), jnp.float32),
            pltpu.SemaphoreType.DMA((NB,)),
            pltpu.SemaphoreType.DMA((NB,)),
        ],
        name=f"sc_gather_{e0}",
    )
    def _gather(x_hbm, senders_hbm, receivers_hbm, xs_hbm, xr_hbm,
                idx_v, bufs, gsems, ssems):
        c = lax.axis_index("c")
        s = lax.axis_index("s")
        wid = s * NC + c
        base = pl.multiple_of(wid * EPW, 8)
        src = pl.multiple_of(e0 + wid * EPW, 8)
        pltpu.sync_copy(senders_hbm.at[pl.ds(src, EPW)], idx_v)
        _run_gather_phase(x_hbm, xs_hbm, idx_v, bufs, gsems, ssems, base)
        pltpu.sync_copy(receivers_hbm.at[pl.ds(src, EPW)], idx_v)
        _run_gather_phase(x_hbm, xr_hbm, idx_v, bufs, gsems, ssems, base)

    return _gather


def _make_scatter(e0):
    @functools.partial(
        pl.kernel,
        out_type=jax.ShapeDtypeStruct((NC, NPAD, D), jnp.float32),
        mesh=_sc_mesh,
        scratch_types=[
            pltpu.VMEM((CHUNK,), jnp.int32),
            pltpu.VMEM((CHUNK,), jnp.int32),
            pltpu.VMEM((2, CHUNK, D), jnp.float32),
            pltpu.VMEM_SHARED((NPAD, D), jnp.float32),
            pltpu.SemaphoreType.DMA((2,)),
            pltpu.SemaphoreType.DMA((2,)),
        ],
        name=f"sc_scatter_{e0}",
    )
    def _scatter(msgs_hbm, receivers_hbm, zeros_hbm, out_hbm,
                 i0, i1, bufs, acc_sh, isems, lsems):
        c = lax.axis_index("c")
        s = lax.axis_index("s")
        ibufs = (i0, i1)
        roff = pl.multiple_of(s * ROWS_PER_SUB, 8)
        pltpu.sync_copy(zeros_hbm, acc_sh.at[pl.ds(roff, ROWS_PER_SUB)])
        plsc.subcore_barrier()

        wid = s * NC + c
        base = wid * EPW
        rbase = e0 + wid * EPW

        def i_desc(t, b):
            off = pl.multiple_of(rbase + t * CHUNK, 8)
            return pltpu.make_async_copy(receivers_hbm.at[pl.ds(off, CHUNK)],
                                         ibufs[b], isems.at[b])

        def l_desc(t, b):
            off = pl.multiple_of(base + t * CHUNK, 8)
            return pltpu.make_async_copy(msgs_hbm.at[pl.ds(off, CHUNK)],
                                         bufs.at[b], lsems.at[b])

        i_desc(0, 0).start()
        l_desc(0, 0).start()

        def outer(o, carry):
            for b in range(2):
                t = o * 2 + b
                i_desc(t, b).wait()
                l_desc(t, b).wait()

                @pl.when(t < NG - 1)
                def _():
                    i_desc(t + 1, 1 - b).start()
                    l_desc(t + 1, 1 - b).start()

                pltpu.sync_copy(bufs.at[b], acc_sh.at[ibufs[b]], add=True)
            return carry

        lax.fori_loop(0, NG // 2, outer, 0)
        if NG % 2:
            t = NG - 1
            b = t % 2
            i_desc(t, b).wait()
            l_desc(t, b).wait()
            pltpu.sync_copy(bufs.at[b], acc_sh.at[ibufs[b]], add=True)
        plsc.subcore_barrier()
        pltpu.sync_copy(acc_sh.at[pl.ds(roff, ROWS_PER_SUB)],
                        out_hbm.at[c].at[pl.ds(roff, ROWS_PER_SUB)])

    return _scatter


_gather_half = [_make_gather(0), _make_gather(EH)]
_scatter_half = [_make_scatter(0), _make_scatter(EH)]


BE = 8000


def _edge_body(xs_ref, xr_ref, ea_ref, w1s_ref, w1r_ref, w1ea_ref, b1_ref,
               w2_ref, b2_ref, out_ref):
    bf = jnp.bfloat16
    mi = (jnp.dot(xs_ref[...].astype(bf), w1s_ref[...],
                  preferred_element_type=jnp.float32)
          + jnp.dot(xr_ref[...].astype(bf), w1r_ref[...],
                    preferred_element_type=jnp.float32)
          + jnp.dot(ea_ref[...], w1ea_ref[...],
                    preferred_element_type=jnp.float32)
          + b1_ref[...])
    h = (mi * jax.nn.sigmoid(mi)).astype(bf)
    out_ref[...] = (jnp.dot(h, w2_ref[...], preferred_element_type=jnp.float32)
                    + b2_ref[...])


def _edge_mlp(xs, xr, ea, w1s, w1r, w1ea, b1, w2, b2):
    grid = (EH // BE,)
    blk = lambda rows, cols: pl.BlockSpec((rows, cols), lambda i: (i, 0))
    full = lambda rows, cols: pl.BlockSpec((rows, cols), lambda i: (0, 0))
    return pl.pallas_call(
        _edge_body,
        grid=grid,
        in_specs=[
            blk(BE, D), blk(BE, D), blk(BE, DE),
            full(D, 2 * D), full(D, 2 * D), full(DE, 2 * D), full(1, 2 * D),
            full(2 * D, D), full(1, D),
        ],
        out_specs=blk(BE, D),
        out_shape=jax.ShapeDtypeStruct((EH, D), jnp.float32),
    )(xs, xr, ea, w1s, w1r, w1ea, b1, w2, b2)


BN = 1000


def _ln(h, g, o):
    mean = jnp.mean(h, axis=-1, keepdims=True)
    var = jnp.mean((h - mean) ** 2, axis=-1, keepdims=True)
    return (h - mean) * jax.lax.rsqrt(var + EPS) * g + o


def _node_body(x_ref, a0_ref, a1_ref, a2_ref, a3_ref, mask_ref, w1x_ref,
               w1a_ref, b1_ref, w2_ref, b2_ref, gm_ref, om_ref, gn_ref,
               on_ref, out_ref):
    mask = mask_ref[...]
    agg = (a0_ref[...] + a1_ref[...] + a2_ref[...] + a3_ref[...]) \
        * INV_MAX_NEIGHBOR
    agg = _ln(agg, gm_ref[...], om_ref[...]) * mask
    x = x_ref[...]
    pre = (jnp.dot(x, w1x_ref[...], preferred_element_type=jnp.float32)
           + jnp.dot(agg, w1a_ref[...], preferred_element_type=jnp.float32)
           + b1_ref[...])
    h2 = pre * jax.nn.sigmoid(pre)
    nf = (jnp.dot(h2, w2_ref[...], preferred_element_type=jnp.float32)
          + b2_ref[...] + x)
    out_ref[...] = _ln(nf, gn_ref[...], on_ref[...]) * mask


def _node_mlp(x, a0, a1, a2, a3, mask, w1x, w1a, b1, w2, b2, gm, om, gn, on):
    grid = (N // BN,)
    blk = lambda rows, cols: pl.BlockSpec((rows, cols), lambda i: (i, 0))
    full = lambda rows, cols: pl.BlockSpec((rows, cols), lambda i: (0, 0))
    return pl.pallas_call(
        _node_body,
        grid=grid,
        in_specs=[
            blk(BN, D), blk(BN, D), blk(BN, D), blk(BN, D), blk(BN, D),
            blk(BN, 1),
            full(D, 2 * D), full(D, 2 * D), full(1, 2 * D),
            full(2 * D, D), full(1, D),
            full(1, D), full(1, D), full(1, D), full(1, D),
        ],
        out_specs=blk(BN, D),
        out_shape=jax.ShapeDtypeStruct((N, D), jnp.float32),
    )(x, a0, a1, a2, a3, mask, w1x, w1a, b1, w2, b2, gm, om, gn, on)


def kernel(x, senders, receivers, edge_attr, nodes_mask,
           W1e, b1e, W2e, b2e, W1n, b1n, W2n, b2n,
           g_msg, o_msg, g_node, o_node):
    bf = jnp.bfloat16
    w1s = W1e[:D].astype(bf)
    w1r = W1e[D:2 * D].astype(bf)
    w1ea = W1e[2 * D:]
    b1 = b1e.reshape(1, 2 * D)
    w2 = W2e.astype(bf)
    b2 = b2e.reshape(1, D)
    zeros = jnp.zeros((ROWS_PER_SUB, D), jnp.float32)

    parts = []
    for half in range(NHALF):
        xs, xr = _gather_half[half](x, senders, receivers)
        ea = lax.slice_in_dim(edge_attr, half * EH, (half + 1) * EH)
        msgs = _edge_mlp(xs, xr, ea, w1s, w1r, w1ea, b1, w2, b2)
        parts.append(_scatter_half[half](msgs, receivers, zeros))

    return _node_mlp(
        x, parts[0][0, :N], parts[0][1, :N], parts[1][0, :N], parts[1][1, :N],
        nodes_mask,
        W1n[:D], W1n[D:2 * D], b1n.reshape(1, 2 * D),
        W2n, b2n.reshape(1, D),
        g_msg.reshape(1, D), o_msg.reshape(1, D),
        g_node.reshape(1, D), o_node.reshape(1, D),
    )


# speedup vs baseline: 1.0056x; 1.0056x over previous
import functools

import jax
import jax.numpy as jnp
from jax import lax
from jax.experimental import pallas as pl
from jax.experimental.pallas import tpu as pltpu
from jax.experimental.pallas import tpu_sc as plsc

N = 10000
E = 320000
D = 128
DE = 16
INV_MAX_NEIGHBOR = 1.0 / 32.0
EPS = 1e-5

NC = 2
NS = 16
NW = NC * NS
NPAD = 10240
ROWS_PER_SUB = NPAD // NS

NHALF = 2
EH = E // NHALF
EPW = EH // NW
CHUNK = 40
NG = EPW // CHUNK
NB = 8
LAG = 4

_sc_mesh = plsc.VectorSubcoreMesh(core_axis_name="c", subcore_axis_name="s")


def _run_gather_phase(table_hbm, dst_hbm, idx_v, bufs, gsems, ssems, base):
    def idx_at(t):
        return idx_v.at[pl.ds(pl.multiple_of(t * CHUNK, 8), CHUNK)]

    def g_desc(t, b):
        return pltpu.make_async_copy(table_hbm.at[idx_at(t)], bufs.at[b],
                                     gsems.at[b])

    def s_desc(t, b):
        off = pl.multiple_of(base + t * CHUNK, 8)
        return pltpu.make_async_copy(bufs.at[b], dst_hbm.at[pl.ds(off, CHUNK)],
                                     ssems.at[b])

    def slot(t, b, first_round):
        if not first_round:
            s_desc(t - NB, b).wait()
        g_desc(t, b).start()
        if not (first_round and b < LAG):
            tl, bl = t - LAG, (b - LAG) % NB
            g_desc(tl, bl).wait()
            s_desc(tl, bl).start()

    for t in range(NB):
        slot(t, t, True)

    def outer(o, carry):
        for b in range(NB):
            slot(o * NB + b, b, False)
        return carry

    lax.fori_loop(1, NG // NB, outer, 0)

    for t in range((NG // NB) * NB, NG):
        slot(t, t % NB, False)
    for t in range(NG - LAG, NG):
        g_desc(t, t % NB).wait()
        s_desc(t, t % NB).start()
    for t in range(NG - NB, NG):
        s_desc(t, t % NB).wait()


def _make_gather(e0):
    @functools.partial(
        pl.kernel,
        out_type=(
            jax.ShapeDtypeStruct((EH, D), jnp.float32),
            jax.ShapeDtypeStruct((EH, D), jnp.float32),
        ),
        mesh=_sc_mesh,
        scratch_types=[
            pltpu.VMEM((EPW,), jnp.int32),
            pltpu.VMEM((NB, CHUNK, D), jnp.float32),
            pltpu.SemaphoreType.DMA((NB,)),
            pltpu.SemaphoreType.DMA((NB,)),
        ],
        name=f"sc_gather_{e0}",
    )
    def _gather(x_hbm, senders_hbm, receivers_hbm, xs_hbm, xr_hbm,
                idx_v, bufs, gsems, ssems):
        c = lax.axis_index("c")
        s = lax.axis_index("s")
        wid = s * NC + c
        base = pl.multiple_of(wid * EPW, 8)
        src = pl.multiple_of(e0 + wid * EPW, 8)
        pltpu.sync_copy(senders_hbm.at[pl.ds(src, EPW)], idx_v)
        _run_gather_phase(x_hbm, xs_hbm, idx_v, bufs, gsems, ssems, base)
        pltpu.sync_copy(receivers_hbm.at[pl.ds(src, EPW)], idx_v)
        _run_gather_phase(x_hbm, xr_hbm, idx_v, bufs, gsems, ssems, base)

    return _gather


def _make_scatter(e0):
    @functools.partial(
        pl.kernel,
        out_type=jax.ShapeDtypeStruct((NC, NPAD, D), jnp.float32),
        mesh=_sc_mesh,
        scratch_types=[
            pltpu.VMEM((CHUNK,), jnp.int32),
            pltpu.VMEM((CHUNK,), jnp.int32),
            pltpu.VMEM((2, CHUNK, D), jnp.float32),
            pltpu.VMEM_SHARED((NPAD, D), jnp.float32),
            pltpu.SemaphoreType.DMA((2,)),
            pltpu.SemaphoreType.DMA((2,)),
        ],
        name=f"sc_scatter_{e0}",
    )
    def _scatter(msgs_hbm, receivers_hbm, zeros_hbm, out_hbm,
                 i0, i1, bufs, acc_sh, isems, lsems):
        c = lax.axis_index("c")
        s = lax.axis_index("s")
        ibufs = (i0, i1)
        roff = pl.multiple_of(s * ROWS_PER_SUB, 8)
        pltpu.sync_copy(zeros_hbm, acc_sh.at[pl.ds(roff, ROWS_PER_SUB)])
        plsc.subcore_barrier()

        wid = s * NC + c
        base = wid * EPW
        rbase = e0 + wid * EPW

        def i_desc(t, b):
            off = pl.multiple_of(rbase + t * CHUNK, 8)
            return pltpu.make_async_copy(receivers_hbm.at[pl.ds(off, CHUNK)],
                                         ibufs[b], isems.at[b])

        def l_desc(t, b):
            off = pl.multiple_of(base + t * CHUNK, 8)
            return pltpu.make_async_copy(msgs_hbm.at[pl.ds(off, CHUNK)],
                                         bufs.at[b], lsems.at[b])

        i_desc(0, 0).start()
        l_desc(0, 0).start()

        def outer(o, carry):
            for b in range(2):
                t = o * 2 + b
                i_desc(t, b).wait()
                l_desc(t, b).wait()

                @pl.when(t < NG - 1)
                def _():
                    i_desc(t + 1, 1 - b).start()
                    l_desc(t + 1, 1 - b).start()

                pltpu.sync_copy(bufs.at[b], acc_sh.at[ibufs[b]], add=True)
            return carry

        lax.fori_loop(0, NG // 2, outer, 0)
        if NG % 2:
            t = NG - 1
            b = t % 2
            i_desc(t, b).wait()
            l_desc(t, b).wait()
            pltpu.sync_copy(bufs.at[b], acc_sh.at[ibufs[b]], add=True)
        plsc.subcore_barrier()
        pltpu.sync_copy(acc_sh.at[pl.ds(roff, ROWS_PER_SUB)],
                        out_hbm.at[c].at[pl.ds(roff, ROWS_PER_SUB)])

    return _scatter


_gather_half = [_make_gather(0), _make_gather(EH)]
_scatter_half = [_make_scatter(0), _make_scatter(EH)]


BE = 4000


def _edge_body(xs_ref, xr_ref, ea_ref, w1s_ref, w1r_ref, w1ea_ref, b1_ref,
               w2_ref, b2_ref, out_ref):
    bf = jnp.bfloat16
    mi = (jnp.dot(xs_ref[...].astype(bf), w1s_ref[...],
                  preferred_element_type=jnp.float32)
          + jnp.dot(xr_ref[...].astype(bf), w1r_ref[...],
                    preferred_element_type=jnp.float32)
          + jnp.dot(ea_ref[...], w1ea_ref[...],
                    preferred_element_type=jnp.float32)
          + b1_ref[...])
    h = (mi * jax.nn.sigmoid(mi)).astype(bf)
    out_ref[...] = (jnp.dot(h, w2_ref[...], preferred_element_type=jnp.float32)
                    + b2_ref[...])


def _edge_mlp(xs, xr, ea, w1s, w1r, w1ea, b1, w2, b2):
    grid = (EH // BE,)
    blk = lambda rows, cols: pl.BlockSpec((rows, cols), lambda i: (i, 0))
    full = lambda rows, cols: pl.BlockSpec((rows, cols), lambda i: (0, 0))
    return pl.pallas_call(
        _edge_body,
        grid=grid,
        in_specs=[
            blk(BE, D), blk(BE, D), blk(BE, DE),
            full(D, 2 * D), full(D, 2 * D), full(DE, 2 * D), full(1, 2 * D),
            full(2 * D, D), full(1, D),
        ],
        out_specs=blk(BE, D),
        out_shape=jax.ShapeDtypeStruct((EH, D), jnp.float32),
    )(xs, xr, ea, w1s, w1r, w1ea, b1, w2, b2)


BN = 1000


def _ln(h, g, o):
    mean = jnp.mean(h, axis=-1, keepdims=True)
    var = jnp.mean((h - mean) ** 2, axis=-1, keepdims=True)
    return (h - mean) * jax.lax.rsqrt(var + EPS) * g + o


def _node_body(x_ref, a0_ref, a1_ref, a2_ref, a3_ref, mask_ref, w1x_ref,
               w1a_ref, b1_ref, w2_ref, b2_ref, gm_ref, om_ref, gn_ref,
               on_ref, out_ref):
    mask = mask_ref[...]
    agg = (a0_ref[...] + a1_ref[...] + a2_ref[...] + a3_ref[...]) \
        * INV_MAX_NEIGHBOR
    agg = _ln(agg, gm_ref[...], om_ref[...]) * mask
    x = x_ref[...]
    pre = (jnp.dot(x, w1x_ref[...], preferred_element_type=jnp.float32)
           + jnp.dot(agg, w1a_ref[...], preferred_element_type=jnp.float32)
           + b1_ref[...])
    h2 = pre * jax.nn.sigmoid(pre)
    nf = (jnp.dot(h2, w2_ref[...], preferred_element_type=jnp.float32)
          + b2_ref[...] + x)
    out_ref[...] = _ln(nf, gn_ref[...], on_ref[...]) * mask


def _node_mlp(x, a0, a1, a2, a3, mask, w1x, w1a, b1, w2, b2, gm, om, gn, on):
    grid = (N // BN,)
    blk = lambda rows, cols: pl.BlockSpec((rows, cols), lambda i: (i, 0))
    full = lambda rows, cols: pl.BlockSpec((rows, cols), lambda i: (0, 0))
    return pl.pallas_call(
        _node_body,
        grid=grid,
        in_specs=[
            blk(BN, D), blk(BN, D), blk(BN, D), blk(BN, D), blk(BN, D),
            blk(BN, 1),
            full(D, 2 * D), full(D, 2 * D), full(1, 2 * D),
            full(2 * D, D), full(1, D),
            full(1, D), full(1, D), full(1, D), full(1, D),
        ],
        out_specs=blk(BN, D),
        out_shape=jax.ShapeDtypeStruct((N, D), jnp.float32),
    )(x, a0, a1, a2, a3, mask, w1x, w1a, b1, w2, b2, gm, om, gn, on)


def kernel(x, senders, receivers, edge_attr, nodes_mask,
           W1e, b1e, W2e, b2e, W1n, b1n, W2n, b2n,
           g_msg, o_msg, g_node, o_node):
    bf = jnp.bfloat16
    w1s = W1e[:D].astype(bf)
    w1r = W1e[D:2 * D].astype(bf)
    w1ea = W1e[2 * D:]
    b1 = b1e.reshape(1, 2 * D)
    w2 = W2e.astype(bf)
    b2 = b2e.reshape(1, D)
    zeros = jnp.zeros((ROWS_PER_SUB, D), jnp.float32)

    parts = []
    for half in range(NHALF):
        xs, xr = _gather_half[half](x, senders, receivers)
        ea = lax.slice_in_dim(edge_attr, half * EH, (half + 1) * EH)
        msgs = _edge_mlp(xs, xr, ea, w1s, w1r, w1ea, b1, w2, b2)
        parts.append(_scatter_half[half](msgs, receivers, zeros))

    return _node_mlp(
        x, parts[0][0, :N], parts[0][1, :N], parts[1][0, :N], parts[1][1, :N],
        nodes_mask,
        W1n[:D], W1n[D:2 * D], b1n.reshape(1, 2 * D),
        W2n, b2n.reshape(1, D),
        g_msg.reshape(1, D), o_msg.reshape(1, D),
        g_node.reshape(1, D), o_node.reshape(1, D),
    )
